# baseline (speedup 1.0000x reference)
import functools

import jax
import jax.numpy as jnp
from jax import lax
from jax.experimental import pallas as pl
from jax.experimental.pallas import tpu as pltpu
from jax.experimental.pallas import tpu_sc as plsc

D = 32
CHUNK = 1600
SPLIT = 8
SUB = CHUNK // SPLIT


@functools.lru_cache(maxsize=None)
def _make(n_rows: int):
  info = plsc.get_sparse_core_info()
  nc, ns = info.num_cores, info.num_subcores
  nw = nc * ns
  rows_per_w = n_rows // nw
  assert rows_per_w * nw == n_rows
  nchunks = rows_per_w // CHUNK
  assert nchunks * CHUNK == rows_per_w and nchunks % 2 == 0
  mesh = plsc.VectorSubcoreMesh(core_axis_name="c", subcore_axis_name="s")

  @functools.partial(
      pl.kernel,
      mesh=mesh,
      out_type=jax.ShapeDtypeStruct((n_rows, D), jnp.float32),
      compiler_params=pltpu.CompilerParams(use_tc_tiling_on_sc=False),
      scratch_types=[
          pltpu.VMEM((2, CHUNK), jnp.int32),
          pltpu.VMEM((2, CHUNK, D), jnp.float32),
          pltpu.SemaphoreType.DMA((2,)),
          pltpu.SemaphoreType.DMA((2,)),
          pltpu.SemaphoreType.DMA((2,)),
      ],
  )
  def gather_kernel(widths_hbm, table_hbm, out_hbm, idx_v, rows_v, sem_i,
                    sem_g, sem_o):
    wid = lax.axis_index("s") * nc + lax.axis_index("c")
    base = wid * rows_per_w

    def start_idx(c, p):
      pltpu.async_copy(widths_hbm.at[pl.ds(base + c * CHUNK, CHUNK)],
                       idx_v.at[p], sem_i.at[p])

    def wait_idx(p):
      pltpu.make_async_copy(widths_hbm.at[pl.ds(base, CHUNK)], idx_v.at[p],
                            sem_i.at[p]).wait()

    def start_gather(p):
      for s in range(SPLIT):
        pltpu.async_copy(table_hbm.at[idx_v.at[p, pl.ds(s * SUB, SUB)]],
                         rows_v.at[p, pl.ds(s * SUB, SUB)], sem_g.at[p])

    def wait_gather(p):
      for s in range(SPLIT):
        pltpu.make_async_copy(table_hbm.at[idx_v.at[p, pl.ds(s * SUB, SUB)]],
                              rows_v.at[p, pl.ds(s * SUB, SUB)],
                              sem_g.at[p]).wait()

    def start_out(c, p):
      pltpu.async_copy(rows_v.at[p],
                       out_hbm.at[pl.ds(base + c * CHUNK, CHUNK)],
                       sem_o.at[p])

    def wait_out(p):
      pltpu.make_async_copy(rows_v.at[p], out_hbm.at[pl.ds(base, CHUNK)],
                            sem_o.at[p]).wait()

    start_idx(0, 0)
    start_idx(1, 1)
    wait_idx(0)
    start_gather(0)

    def body(gg, carry):
      for p in range(2):
        c = gg * 2 + p
        q = 1 - p

        @pl.when(c + 1 < nchunks)
        def _():
          wait_idx(q)

          @pl.when(c + 1 >= 2)
          def _():
            wait_out(q)

          start_gather(q)

        wait_gather(p)
        start_out(c, p)

        @pl.when(c + 2 < nchunks)
        def _():
          start_idx_c2 = c + 2
          pltpu.async_copy(
              widths_hbm.at[pl.ds(base + start_idx_c2 * CHUNK, CHUNK)],
              idx_v.at[p], sem_i.at[p])

      return carry

    lax.fori_loop(0, nchunks // 2, body, 0)
    wait_out(0)
    wait_out(1)

  return gather_kernel


def kernel(widths, table):
  b, h = widths.shape
  n_rows = b * h
  flat = widths.reshape(n_rows)
  out = _make(n_rows)(flat, table)
  return out.reshape(b, h, D)


# speedup vs baseline: 1.0008x; 1.0008x over previous
import functools

import jax
import jax.numpy as jnp
from jax import lax
from jax.experimental import pallas as pl
from jax.experimental.pallas import tpu as pltpu
from jax.experimental.pallas import tpu_sc as plsc

D = 32
CHUNK = 1600
SPLIT = 2
SUB = CHUNK // SPLIT


@functools.lru_cache(maxsize=None)
def _make(n_rows: int):
  info = plsc.get_sparse_core_info()
  nc, ns = info.num_cores, info.num_subcores
  nw = nc * ns
  rows_per_w = n_rows // nw
  assert rows_per_w * nw == n_rows
  nchunks = rows_per_w // CHUNK
  assert nchunks * CHUNK == rows_per_w and nchunks % 2 == 0
  mesh = plsc.VectorSubcoreMesh(core_axis_name="c", subcore_axis_name="s")

  @functools.partial(
      pl.kernel,
      mesh=mesh,
      out_type=jax.ShapeDtypeStruct((n_rows, D), jnp.float32),
      compiler_params=pltpu.CompilerParams(use_tc_tiling_on_sc=False),
      scratch_types=[
          pltpu.VMEM((2, CHUNK), jnp.int32),
          pltpu.VMEM((2, CHUNK, D), jnp.float32),
          pltpu.SemaphoreType.DMA((2,)),
          pltpu.SemaphoreType.DMA((2,)),
          pltpu.SemaphoreType.DMA((2,)),
      ],
  )
  def gather_kernel(widths_hbm, table_hbm, out_hbm, idx_v, rows_v, sem_i,
                    sem_g, sem_o):
    wid = lax.axis_index("s") * nc + lax.axis_index("c")
    base = wid * rows_per_w

    def start_idx(c, p):
      pltpu.async_copy(widths_hbm.at[pl.ds(base + c * CHUNK, CHUNK)],
                       idx_v.at[p], sem_i.at[p])

    def wait_idx(p):
      pltpu.make_async_copy(widths_hbm.at[pl.ds(base, CHUNK)], idx_v.at[p],
                            sem_i.at[p]).wait()

    def start_gather(p):
      for s in range(SPLIT):
        pltpu.async_copy(table_hbm.at[idx_v.at[p, pl.ds(s * SUB, SUB)]],
                         rows_v.at[p, pl.ds(s * SUB, SUB)], sem_g.at[p])

    def wait_gather(p):
      for s in range(SPLIT):
        pltpu.make_async_copy(table_hbm.at[idx_v.at[p, pl.ds(s * SUB, SUB)]],
                              rows_v.at[p, pl.ds(s * SUB, SUB)],
                              sem_g.at[p]).wait()

    def start_out(c, p):
      pltpu.async_copy(rows_v.at[p],
                       out_hbm.at[pl.ds(base + c * CHUNK, CHUNK)],
                       sem_o.at[p])

    def wait_out(p):
      pltpu.make_async_copy(rows_v.at[p], out_hbm.at[pl.ds(base, CHUNK)],
                            sem_o.at[p]).wait()

    start_idx(0, 0)
    start_idx(1, 1)
    wait_idx(0)
    start_gather(0)

    def body(gg, carry):
      for p in range(2):
        c = gg * 2 + p
        q = 1 - p

        @pl.when(c + 1 < nchunks)
        def _():
          wait_idx(q)

          @pl.when(c + 1 >= 2)
          def _():
            wait_out(q)

          start_gather(q)

        wait_gather(p)
        start_out(c, p)

        @pl.when(c + 2 < nchunks)
        def _():
          start_idx_c2 = c + 2
          pltpu.async_copy(
              widths_hbm.at[pl.ds(base + start_idx_c2 * CHUNK, CHUNK)],
              idx_v.at[p], sem_i.at[p])

      return carry

    lax.fori_loop(0, nchunks // 2, body, 0)
    wait_out(0)
    wait_out(1)

  return gather_kernel


def kernel(widths, table):
  b, h = widths.shape
  n_rows = b * h
  flat = widths.reshape(n_rows)
  out = _make(n_rows)(flat, table)
  return out.reshape(b, h, D)
